# baseline (speedup 1.0000x reference)
import functools

import jax
import jax.numpy as jnp
from jax import lax
from jax.experimental import pallas as pl
from jax.experimental.pallas import tpu as pltpu
from jax.experimental.pallas import tpu_sc as plsc

_STORAGE_SIZE = 100000
_CAMS = 8
_FEATS = 7
_BATCH = 16384

_NW = 32
_FRAMES_W = _BATCH // _NW
_LOOK_W = _FRAMES_W * _CAMS
_CHUNK = 128
_NCHUNK = _LOOK_W // _CHUNK
_L = 16
_NROW8 = _STORAGE_SIZE * _CAMS
_PLANE8 = _STORAGE_SIZE


def _sc_body(frame_hbm, tablet_hbm, rot_hbm, trans_hbm, focal_hbm,
             fidx_v, idx3d, s_st, rows7, rot_st, trans_st, focal_st, sem):
    wid = lax.axis_index("s") * 2 + lax.axis_index("c")

    iota = lax.iota(jnp.int32, _L)
    sel = lax.shift_right_logical(iota, 3)
    camoff = (iota & 7) * _STORAGE_SIZE

    pltpu.sync_copy(frame_hbm.at[pl.ds(wid * _FRAMES_W, _FRAMES_W)], fidx_v)

    def build(c, carry):
        for u in range(8):
            p0 = c * _CHUNK + u * _L
            frames = lax.shift_right_logical(p0, 3) + sel
            fvals = plsc.load_gather(fidx_v, [frames])
            r = fvals + camoff
            q0 = lax.shift_right_logical(r, 3)
            s_st[pl.ds(p0, _L)] = r & 7
            for j in range(_FEATS):
                idx3d[c, j, pl.ds(u * _L, _L)] = q0 + j * _PLANE8
        return carry

    lax.fori_loop(0, _NCHUNK, build, 0)

    def chunk(c, carry):
        for j in range(_FEATS):
            pltpu.async_copy(tablet_hbm.at[idx3d.at[c, j]], rows7.at[j], sem)
        for j in range(_FEATS):
            pltpu.make_async_copy(
                tablet_hbm.at[idx3d.at[c, j]], rows7.at[j], sem).wait()
        c16 = c * 16
        cbase = c * _CHUNK
        for k in range(24):
            q = k * _L + iota
            l = lax.shift_right_logical(q * 171, 9)
            jv = q - l * 3
            sv = plsc.load_gather(s_st, [cbase + l])
            i0 = lax.shift_right_logical(l, 3) + c16
            i1 = l & 7
            rotv = plsc.load_gather(rows7, [jv, l, sv])
            plsc.store_scatter(rot_st, [i0, i1, jv], rotv)
            transv = plsc.load_gather(rows7, [jv + 3, l, sv]) * 10.0
            plsc.store_scatter(trans_st, [i0, i1, jv], transv)
        col6 = (iota & 0) + 6
        for k in range(8):
            q = k * _L + iota
            sv = plsc.load_gather(s_st, [cbase + q])
            focv = plsc.load_gather(rows7, [col6, q, sv]) * 1000.0
            plsc.store_scatter(
                focal_st,
                [lax.shift_right_logical(q, 3) + c16, q & 7], focv)
        return carry

    lax.fori_loop(0, _NCHUNK, chunk, 0)

    f0 = wid * _FRAMES_W
    pltpu.sync_copy(rot_st, rot_hbm.at[pl.ds(f0, _FRAMES_W)])
    pltpu.sync_copy(trans_st, trans_hbm.at[pl.ds(f0, _FRAMES_W)])
    pltpu.sync_copy(focal_st, focal_hbm.at[pl.ds(f0, _FRAMES_W)])


@jax.jit
def _sc_call(frame_indexes, storage):
    mesh = plsc.VectorSubcoreMesh(core_axis_name="c", subcore_axis_name="s")
    f = functools.partial(
        pl.kernel,
        mesh=mesh,
        out_type=[
            jax.ShapeDtypeStruct((_BATCH, _CAMS, 3), jnp.float32),
            jax.ShapeDtypeStruct((_BATCH, _CAMS, 3), jnp.float32),
            jax.ShapeDtypeStruct((_BATCH, _CAMS), jnp.float32),
        ],
        scratch_types=[
            pltpu.VMEM((_FRAMES_W,), jnp.int32),
            pltpu.VMEM((_NCHUNK, _FEATS, _CHUNK), jnp.int32),
            pltpu.VMEM((_LOOK_W,), jnp.int32),
            pltpu.VMEM((_FEATS, _CHUNK, 8), jnp.float32),
            pltpu.VMEM((_FRAMES_W, _CAMS, 3), jnp.float32),
            pltpu.VMEM((_FRAMES_W, _CAMS, 3), jnp.float32),
            pltpu.VMEM((_FRAMES_W, _CAMS), jnp.float32),
            pltpu.SemaphoreType.DMA,
        ],
        compiler_params=pltpu.CompilerParams(
            use_tc_tiling_on_sc=False, needs_layout_passes=False),
    )(_sc_body)
    padded = jnp.concatenate(
        [storage, jnp.zeros((_STORAGE_SIZE * _CAMS, 1), jnp.float32)], axis=1)
    return f(frame_indexes, padded.T.reshape(_NROW8, 8))


def kernel(frame_indexes, storage):
    rot, trans, focal = _sc_call(frame_indexes, storage)
    return (rot, trans, focal)


# speedup vs baseline: 1.9427x; 1.9427x over previous
import functools

import jax
import jax.numpy as jnp
from jax import lax
from jax.experimental import pallas as pl
from jax.experimental.pallas import tpu as pltpu
from jax.experimental.pallas import tpu_sc as plsc

_STORAGE_SIZE = 100000
_CAMS = 8
_FEATS = 7
_BATCH = 16384

_NW = 32
_FRAMES_W = _BATCH // _NW
_LOOK_W = _FRAMES_W * _CAMS
_CHUNK = 128
_NCHUNK = _LOOK_W // _CHUNK
_L = 16
_PROWS = _STORAGE_SIZE * _CAMS // 8


def _sc_body(frame_hbm, p0_hbm, p1_hbm, p2_hbm, p3_hbm, p4_hbm, p5_hbm,
             p6_hbm, rot_hbm, trans_hbm, focal_hbm,
             fidx_v, idx2d, s_st, rows7, rot_st, trans_st, focal_st, sem):
    wid = lax.axis_index("s") * 2 + lax.axis_index("c")
    planes = (p0_hbm, p1_hbm, p2_hbm, p3_hbm, p4_hbm, p5_hbm, p6_hbm)

    iota = lax.iota(jnp.int32, _L)
    sel = lax.shift_right_logical(iota, 3)
    camoff = (iota & 7) * _STORAGE_SIZE

    pltpu.sync_copy(frame_hbm.at[pl.ds(wid * _FRAMES_W, _FRAMES_W)], fidx_v)

    def build(c, carry):
        for u in range(8):
            p0 = c * _CHUNK + u * _L
            frames = lax.shift_right_logical(p0, 3) + sel
            fvals = plsc.load_gather(fidx_v, [frames])
            r = fvals + camoff
            s_st[pl.ds(p0, _L)] = r & 7
            idx2d[c, pl.ds(u * _L, _L)] = lax.shift_right_logical(r, 3)
        return carry

    lax.fori_loop(0, _NCHUNK, build, 0)

    def chunk(c, carry):
        for j in range(_FEATS):
            pltpu.async_copy(planes[j].at[idx2d.at[c]], rows7.at[j], sem)
        for j in range(_FEATS):
            pltpu.make_async_copy(
                planes[j].at[idx2d.at[c]], rows7.at[j], sem).wait()
        c16 = c * 16
        cbase = c * _CHUNK
        for k in range(24):
            q = k * _L + iota
            l = lax.shift_right_logical(q * 171, 9)
            jv = q - l * 3
            sv = plsc.load_gather(s_st, [cbase + l])
            i0 = lax.shift_right_logical(l, 3) + c16
            i1 = l & 7
            rotv = plsc.load_gather(rows7, [jv, l, sv])
            plsc.store_scatter(rot_st, [i0, i1, jv], rotv)
            transv = plsc.load_gather(rows7, [jv + 3, l, sv]) * 10.0
            plsc.store_scatter(trans_st, [i0, i1, jv], transv)
        col6 = (iota & 0) + 6
        for k in range(8):
            q = k * _L + iota
            sv = plsc.load_gather(s_st, [cbase + q])
            focv = plsc.load_gather(rows7, [col6, q, sv]) * 1000.0
            plsc.store_scatter(
                focal_st,
                [lax.shift_right_logical(q, 3) + c16, q & 7], focv)
        return carry

    lax.fori_loop(0, _NCHUNK, chunk, 0)

    f0 = wid * _FRAMES_W
    pltpu.sync_copy(rot_st, rot_hbm.at[pl.ds(f0, _FRAMES_W)])
    pltpu.sync_copy(trans_st, trans_hbm.at[pl.ds(f0, _FRAMES_W)])
    pltpu.sync_copy(focal_st, focal_hbm.at[pl.ds(f0, _FRAMES_W)])


@jax.jit
def _sc_call(frame_indexes, storage):
    mesh = plsc.VectorSubcoreMesh(core_axis_name="c", subcore_axis_name="s")
    f = functools.partial(
        pl.kernel,
        mesh=mesh,
        out_type=[
            jax.ShapeDtypeStruct((_BATCH, _CAMS, 3), jnp.float32),
            jax.ShapeDtypeStruct((_BATCH, _CAMS, 3), jnp.float32),
            jax.ShapeDtypeStruct((_BATCH, _CAMS), jnp.float32),
        ],
        scratch_types=[
            pltpu.VMEM((_FRAMES_W,), jnp.int32),
            pltpu.VMEM((_NCHUNK, _CHUNK), jnp.int32),
            pltpu.VMEM((_LOOK_W,), jnp.int32),
            pltpu.VMEM((_FEATS, _CHUNK, 8), jnp.float32),
            pltpu.VMEM((_FRAMES_W, _CAMS, 3), jnp.float32),
            pltpu.VMEM((_FRAMES_W, _CAMS, 3), jnp.float32),
            pltpu.VMEM((_FRAMES_W, _CAMS), jnp.float32),
            pltpu.SemaphoreType.DMA,
        ],
        compiler_params=pltpu.CompilerParams(
            use_tc_tiling_on_sc=False, needs_layout_passes=False),
    )(_sc_body)
    planes = [storage[:, j].reshape(_PROWS, 8) for j in range(_FEATS)]
    return f(frame_indexes, *planes)


def kernel(frame_indexes, storage):
    rot, trans, focal = _sc_call(frame_indexes, storage)
    return (rot, trans, focal)


# speedup vs baseline: 3.3919x; 1.7460x over previous
import functools

import jax
import jax.numpy as jnp
from jax import lax
from jax.experimental import pallas as pl
from jax.experimental.pallas import tpu as pltpu
from jax.experimental.pallas import tpu_sc as plsc

_STORAGE_SIZE = 100000
_CAMS = 8
_FEATS = 7
_BATCH = 16384

_NW = 32
_FRAMES_W = _BATCH // _NW
_LOOK_W = _FRAMES_W * _CAMS
_CHUNK = 128
_NCHUNK = _LOOK_W // _CHUNK
_L = 16
_PROWS = _STORAGE_SIZE * _CAMS // 8


def _sc_body(frame_hbm, p0_hbm, p1_hbm, p2_hbm, p3_hbm, p4_hbm, p5_hbm,
             p6_hbm, rot_hbm, trans_hbm, focal_hbm,
             fidx_v, idx2d, s_st, rows7, rot_st, trans_st, focal_st, sem):
    wid = lax.axis_index("s") * 2 + lax.axis_index("c")
    planes = (p0_hbm, p1_hbm, p2_hbm, p3_hbm, p4_hbm, p5_hbm, p6_hbm)

    iota = lax.iota(jnp.int32, _L)
    sel = lax.shift_right_logical(iota, 3)
    camoff = (iota & 7) * _STORAGE_SIZE

    pltpu.sync_copy(frame_hbm.at[pl.ds(wid * _FRAMES_W, _FRAMES_W)], fidx_v)

    def build(c, carry):
        for u in range(8):
            p0 = c * _CHUNK + u * _L
            frames = lax.shift_right_logical(p0, 3) + sel
            fvals = plsc.load_gather(fidx_v, [frames])
            r = fvals + camoff
            s_st[pl.ds(p0, _L)] = r & 7
            idx2d[c, pl.ds(u * _L, _L)] = lax.shift_right_logical(r, 3)
        return carry

    lax.fori_loop(0, _NCHUNK, build, 0)

    def chunk(c, carry):
        for j in range(_FEATS):
            pltpu.async_copy(planes[j].at[idx2d.at[c]], rows7.at[j], sem)
        for j in range(_FEATS):
            pltpu.make_async_copy(
                planes[j].at[idx2d.at[c]], rows7.at[j], sem).wait()
        c16 = c * 16
        cbase = c * _CHUNK
        for k in range(24):
            q = k * _L + iota
            l = lax.shift_right_logical(q * 171, 9)
            jv = q - l * 3
            sv = plsc.load_gather(s_st, [cbase + l])
            fl = lax.shift_right_logical(l, 3) + c16
            fhi = lax.shift_right_logical(fl, 7)
            flo = fl & 127
            i1 = l & 7
            rotv = plsc.load_gather(rows7, [jv, l, sv])
            plsc.store_scatter(rot_st, [jv, fhi, i1, flo], rotv)
            transv = plsc.load_gather(rows7, [jv + 3, l, sv]) * 10.0
            plsc.store_scatter(trans_st, [jv, fhi, i1, flo], transv)
        col6 = (iota & 0) + 6
        for k in range(8):
            q = k * _L + iota
            sv = plsc.load_gather(s_st, [cbase + q])
            focv = plsc.load_gather(rows7, [col6, q, sv]) * 1000.0
            fl = lax.shift_right_logical(q, 3) + c16
            plsc.store_scatter(
                focal_st,
                [lax.shift_right_logical(fl, 7), q & 7, fl & 127], focv)
        return carry

    lax.fori_loop(0, _NCHUNK, chunk, 0)

    b0 = wid * (_FRAMES_W // 128)
    for j in range(3):
        pltpu.sync_copy(rot_st.at[j], rot_hbm.at[j, pl.ds(b0, 4)])
        pltpu.sync_copy(trans_st.at[j], trans_hbm.at[j, pl.ds(b0, 4)])
    pltpu.sync_copy(focal_st, focal_hbm.at[pl.ds(b0, 4)])


@jax.jit
def _sc_call(frame_indexes, storage):
    mesh = plsc.VectorSubcoreMesh(core_axis_name="c", subcore_axis_name="s")
    f = functools.partial(
        pl.kernel,
        mesh=mesh,
        out_type=[
            jax.ShapeDtypeStruct((3, _BATCH // 128, _CAMS, 128), jnp.float32),
            jax.ShapeDtypeStruct((3, _BATCH // 128, _CAMS, 128), jnp.float32),
            jax.ShapeDtypeStruct((_BATCH // 128, _CAMS, 128), jnp.float32),
        ],
        scratch_types=[
            pltpu.VMEM((_FRAMES_W,), jnp.int32),
            pltpu.VMEM((_NCHUNK, _CHUNK), jnp.int32),
            pltpu.VMEM((_LOOK_W,), jnp.int32),
            pltpu.VMEM((_FEATS, _CHUNK, 8), jnp.float32),
            pltpu.VMEM((3, _FRAMES_W // 128, _CAMS, 128), jnp.float32),
            pltpu.VMEM((3, _FRAMES_W // 128, _CAMS, 128), jnp.float32),
            pltpu.VMEM((_FRAMES_W // 128, _CAMS, 128), jnp.float32),
            pltpu.SemaphoreType.DMA,
        ],
        compiler_params=pltpu.CompilerParams(
            use_tc_tiling_on_sc=False, needs_layout_passes=False),
    )(_sc_body)
    planes = [storage[:, j].reshape(_PROWS, 8) for j in range(_FEATS)]
    return f(frame_indexes, *planes)


def kernel(frame_indexes, storage):
    rot4, trans4, focal3 = _sc_call(frame_indexes, storage)
    rot = rot4.transpose(1, 3, 2, 0).reshape(_BATCH, _CAMS, 3)
    trans = trans4.transpose(1, 3, 2, 0).reshape(_BATCH, _CAMS, 3)
    focal = focal3.transpose(0, 2, 1).reshape(_BATCH, _CAMS)
    return (rot, trans, focal)


# speedup vs baseline: 4.0514x; 1.1944x over previous
import functools

import jax
import jax.numpy as jnp
from jax import lax
from jax.experimental import pallas as pl
from jax.experimental.pallas import tpu as pltpu
from jax.experimental.pallas import tpu_sc as plsc

_STORAGE_SIZE = 100000
_CAMS = 8
_FEATS = 7
_BATCH = 16384

_NW = 32
_FRAMES_W = _BATCH // _NW
_LOOK_W = _FRAMES_W * _CAMS
_CHUNK = 128
_NCHUNK = _LOOK_W // _CHUNK
_L = 16
_PROWS = _STORAGE_SIZE * _CAMS // 8


def _sc_body(frame_hbm, p0_hbm, p1_hbm, p2_hbm, p3_hbm, p4_hbm, p5_hbm,
             p6_hbm, rot_hbm, trans_hbm, focal_hbm,
             fidx_v, idx2d, s_st, rows7a, rows7b, rot_st, trans_st, focal_st,
             sema, semb):
    wid = lax.axis_index("s") * 2 + lax.axis_index("c")
    planes = (p0_hbm, p1_hbm, p2_hbm, p3_hbm, p4_hbm, p5_hbm, p6_hbm)

    iota = lax.iota(jnp.int32, _L)
    sel = lax.shift_right_logical(iota, 3)
    camoff = (iota & 7) * _STORAGE_SIZE

    pltpu.sync_copy(frame_hbm.at[pl.ds(wid * _FRAMES_W, _FRAMES_W)], fidx_v)

    def build(c, carry):
        for u in range(8):
            p0 = c * _CHUNK + u * _L
            frames = lax.shift_right_logical(p0, 3) + sel
            fvals = plsc.load_gather(fidx_v, [frames])
            r = fvals + camoff
            s_st[pl.ds(p0, _L)] = r & 7
            idx2d[c, pl.ds(u * _L, _L)] = lax.shift_right_logical(r, 3)
        return carry

    lax.fori_loop(0, _NCHUNK, build, 0)

    def fire(c, buf, sem):
        for j in range(_FEATS):
            pltpu.async_copy(planes[j].at[idx2d.at[c]], buf.at[j], sem)

    def drain(c, buf, sem):
        for j in range(_FEATS):
            pltpu.make_async_copy(
                planes[j].at[idx2d.at[c]], buf.at[j], sem).wait()

    def split(c, rows7):
        c16 = c * 16
        cbase = c * _CHUNK
        for k in range(24):
            q = k * _L + iota
            l = lax.shift_right_logical(q * 171, 9)
            jv = q - l * 3
            sv = plsc.load_gather(s_st, [cbase + l])
            fl = lax.shift_right_logical(l, 3) + c16
            fhi = lax.shift_right_logical(fl, 7)
            flo = fl & 127
            i1 = l & 7
            rotv = plsc.load_gather(rows7, [jv, l, sv])
            plsc.store_scatter(rot_st, [jv, fhi, i1, flo], rotv)
            transv = plsc.load_gather(rows7, [jv + 3, l, sv]) * 10.0
            plsc.store_scatter(trans_st, [jv, fhi, i1, flo], transv)
        col6 = (iota & 0) + 6
        for k in range(8):
            q = k * _L + iota
            sv = plsc.load_gather(s_st, [cbase + q])
            focv = plsc.load_gather(rows7, [col6, q, sv]) * 1000.0
            fl = lax.shift_right_logical(q, 3) + c16
            plsc.store_scatter(
                focal_st,
                [lax.shift_right_logical(fl, 7), q & 7, fl & 127], focv)

    fire(0, rows7a, sema)

    def pair(i, carry):
        ca = 2 * i
        fire(ca + 1, rows7b, semb)
        drain(ca, rows7a, sema)
        split(ca, rows7a)

        @pl.when(i < _NCHUNK // 2 - 1)
        def _():
            fire(ca + 2, rows7a, sema)

        drain(ca + 1, rows7b, semb)
        split(ca + 1, rows7b)
        return carry

    lax.fori_loop(0, _NCHUNK // 2, pair, 0)

    b0 = wid * (_FRAMES_W // 128)
    for j in range(3):
        pltpu.sync_copy(rot_st.at[j], rot_hbm.at[j, pl.ds(b0, 4)])
        pltpu.sync_copy(trans_st.at[j], trans_hbm.at[j, pl.ds(b0, 4)])
    pltpu.sync_copy(focal_st, focal_hbm.at[pl.ds(b0, 4)])


@jax.jit
def _sc_call(frame_indexes, storage):
    mesh = plsc.VectorSubcoreMesh(core_axis_name="c", subcore_axis_name="s")
    f = functools.partial(
        pl.kernel,
        mesh=mesh,
        out_type=[
            jax.ShapeDtypeStruct((3, _BATCH // 128, _CAMS, 128), jnp.float32),
            jax.ShapeDtypeStruct((3, _BATCH // 128, _CAMS, 128), jnp.float32),
            jax.ShapeDtypeStruct((_BATCH // 128, _CAMS, 128), jnp.float32),
        ],
        scratch_types=[
            pltpu.VMEM((_FRAMES_W,), jnp.int32),
            pltpu.VMEM((_NCHUNK, _CHUNK), jnp.int32),
            pltpu.VMEM((_LOOK_W,), jnp.int32),
            pltpu.VMEM((_FEATS, _CHUNK, 8), jnp.float32),
            pltpu.VMEM((_FEATS, _CHUNK, 8), jnp.float32),
            pltpu.VMEM((3, _FRAMES_W // 128, _CAMS, 128), jnp.float32),
            pltpu.VMEM((3, _FRAMES_W // 128, _CAMS, 128), jnp.float32),
            pltpu.VMEM((_FRAMES_W // 128, _CAMS, 128), jnp.float32),
            pltpu.SemaphoreType.DMA,
            pltpu.SemaphoreType.DMA,
        ],
        compiler_params=pltpu.CompilerParams(
            use_tc_tiling_on_sc=False, needs_layout_passes=False),
    )(_sc_body)
    planes = [storage[:, j].reshape(_PROWS, 8) for j in range(_FEATS)]
    return f(frame_indexes, *planes)


def kernel(frame_indexes, storage):
    rot4, trans4, focal3 = _sc_call(frame_indexes, storage)
    rot = rot4.transpose(1, 3, 2, 0).reshape(_BATCH, _CAMS, 3)
    trans = trans4.transpose(1, 3, 2, 0).reshape(_BATCH, _CAMS, 3)
    focal = focal3.transpose(0, 2, 1).reshape(_BATCH, _CAMS)
    return (rot, trans, focal)
